# baseline (speedup 1.0000x reference)
import jax
import jax.numpy as jnp
from jax import lax
from jax.experimental import pallas as pl
from jax.experimental.pallas import tpu as pltpu

F32 = jnp.float32
BF16 = jnp.bfloat16
HIGHEST = lax.Precision.HIGHEST

_PAR = pltpu.CompilerParams(dimension_semantics=("parallel",))


def _shift2d(v, sy, sx):
    H, W, C = v.shape
    core = v[max(0, sy):H + min(0, sy), max(0, sx):W + min(0, sx), :]
    return jnp.pad(core, ((max(0, -sy), max(0, sy)),
                          (max(0, -sx), max(0, sx)), (0, 0)))


def _enc1_kernel(l_ref, w_ref, b_ref, o_ref):
    L = l_ref[0].reshape(4096, 48).astype(BF16)
    acc = jnp.dot(L, w_ref[...], preferred_element_type=F32)
    o_ref[0] = jnp.maximum(acc + b_ref[...], 0.0).reshape(64, 64, 128)


def _enc2_kernel(x_ref, w_ref, b_ref, o_ref):
    X = x_ref[0]
    acc = None
    i = 0
    for ky in range(4):
        dy, jy = divmod(ky, 2)
        for dx in range(2):
            t = X[dy:dy + 32, dx:dx + 32, jy * 256:(jy + 1) * 256]
            p = jnp.dot(t.reshape(1024, 256).astype(BF16), w_ref[i],
                        preferred_element_type=F32)
            acc = p if acc is None else acc + p
            i += 1
    o_ref[0] = jnp.maximum(acc + b_ref[...], 0.0).reshape(32, 32, 256)


def _enc3_vq_kernel(x_ref, w_ref, b_ref, embT_ref, emb_ref,
                    idx_ref, q_ref, loss_ref):
    X = x_ref[0]
    Xp = jnp.pad(X, ((1, 1), (1, 1), (0, 0)))
    acc = None
    for i in range(9):
        ky, kx = divmod(i, 3)
        t = Xp[ky:ky + 32, kx:kx + 32, :].reshape(1024, 256).astype(BF16)
        p = jnp.dot(t, w_ref[i], preferred_element_type=F32)
        acc = p if acc is None else acc + p
    z = acc + b_ref[...]

    A = jnp.sum(z * z, axis=1, keepdims=True)
    embT = embT_ref[...]
    Brow = jnp.sum(embT * embT, axis=0, keepdims=True)
    C = jnp.dot(z.astype(BF16), embT.astype(BF16),
                preferred_element_type=F32)
    dists = (A + Brow) - 2.0 * C
    m = jnp.min(dists, axis=1, keepdims=True)
    iota = lax.broadcasted_iota(jnp.int32, (1024, 512), 1)
    am = jnp.min(jnp.where(dists == m, iota, 512), axis=1).astype(jnp.int32)
    idx_ref[0] = am[None, :]

    onehot = (iota == am[:, None]).astype(F32)
    q = jnp.dot(onehot, emb_ref[...], preferred_element_type=F32,
                precision=HIGHEST)
    q_ref[0] = q.reshape(32, 32, 256).astype(BF16)
    diff = q - z
    loss_ref[0] = jnp.sum(diff * diff, axis=0, keepdims=True)


def _conv3x3_bf16(v, w_ref, b):
    vp = jnp.pad(v, ((1, 1), (1, 1), (0, 0)))
    acc = None
    for i in range(9):
        ky, kx = divmod(i, 3)
        t = vp[ky:ky + 32, kx:kx + 32, :].reshape(1024, 256)
        p = jnp.dot(t, w_ref[i], preferred_element_type=F32)
        acc = p if acc is None else acc + p
    return acc + b


def _dec_res_kernel(x_ref, wd_ref, bd_ref, w1a_ref, b1a_ref, w1b_ref, b1b_ref,
                    w2a_ref, b2a_ref, w2b_ref, b2b_ref, o_ref):
    X = x_ref[0]
    h = _conv3x3_bf16(X, wd_ref, bd_ref[...])
    for wa, ba, wb, bb in ((w1a_ref, b1a_ref, w1b_ref, b1b_ref),
                           (w2a_ref, b2a_ref, w2b_ref, b2b_ref)):
        r = jnp.maximum(h, 0.0).astype(BF16).reshape(32, 32, 256)
        t = _conv3x3_bf16(r, wa, ba[...])
        t = jnp.maximum(t, 0.0).astype(BF16)
        t = jnp.dot(t, wb[...], preferred_element_type=F32) + bb[...]
        h = h + t
    o_ref[0] = h.astype(BF16).reshape(32, 32, 256)


_PH = ((( 0, 1), (-1, 3)), ((1, 0), (0, 2)))


def _dect1_kernel(x_ref, w_ref, b_ref, o_ref):
    X = x_ref[0]
    i = 0
    for py in range(2):
        for px in range(2):
            acc = None
            for (sy, _ky) in _PH[py]:
                for (sx, _kx) in _PH[px]:
                    t = _shift2d(X, sy, sx).reshape(1024, 256)
                    p = jnp.dot(t, w_ref[i], preferred_element_type=F32)
                    acc = p if acc is None else acc + p
                    i += 1
            y = jnp.maximum(acc + b_ref[...], 0.0)
            o_ref[0, py * 2 + px] = y.astype(BF16).reshape(32, 32, 256)


def _dect2_kernel(x_ref, w_ref, b_ref, o_ref):
    X = x_ref[0]
    acc = None
    for i in range(9):
        sy, sx = divmod(i, 3)
        sy -= 1
        sx -= 1
        t = _shift2d(X, sy, sx).reshape(4096, 256)
        p = jnp.dot(t, w_ref[i], preferred_element_type=F32)
        acc = p if acc is None else acc + p
    y = jax.nn.sigmoid(acc + b_ref[...])
    o_ref[0] = y.reshape(64, 64, 16)


def _s2d(x_nhwc):
    B, H, W, C = x_nhwc.shape
    xp = jnp.pad(x_nhwc, ((0, 0), (1, 1), (1, 1), (0, 0)))
    x2 = xp.reshape(B, (H + 2) // 2, 2, (W + 2) // 2, 2, C)
    return x2.transpose(0, 1, 3, 2, 4, 5).reshape(B, (H + 2) // 2, (W + 2) // 2, 4 * C)


def _full(shape):
    nd = len(shape)
    return pl.BlockSpec(shape, lambda b: (0,) * nd)


def _per_img(shape):
    nd = len(shape)
    return pl.BlockSpec((1,) + shape[1:], lambda b: (b,) + (0,) * (nd - 1))


def kernel(x, emb, enc_w1, enc_b1, enc_w2, enc_b2, enc_w3, enc_b3,
           dec_w1, dec_b1, r1_w1, r1_b1, r1_w2, r1_b2,
           r2_w1, r2_b1, r2_w2, r2_b2, dect_w1, dect_b1, dect_w2, dect_b2):
    B = x.shape[0]

    x2 = _s2d(x.transpose(0, 2, 3, 1)).reshape(B, 65, 65, 2, 2, 3)
    parts = []
    wparts = []
    for ky in range(4):
        dy, jy = divmod(ky, 2)
        for kx in range(4):
            dx, jx = divmod(kx, 2)
            parts.append(x2[:, dy:dy + 64, dx:dx + 64, jy, jx, :])
            wparts.append(enc_w1[:, :, ky, kx].T)
    L = jnp.concatenate(parts, axis=-1)
    W48 = jnp.concatenate(wparts, axis=0).astype(BF16)
    h1 = pl.pallas_call(
        _enc1_kernel,
        grid=(B,),
        in_specs=[_per_img((B, 64, 64, 48)), _full((48, 128)), _full((1, 128))],
        out_specs=_per_img((B, 64, 64, 128)),
        out_shape=jax.ShapeDtypeStruct((B, 64, 64, 128), F32),
        compiler_params=_PAR,
    )(L, W48, enc_b1[None, :])

    X2b = _s2d(h1)
    w2stack = jnp.stack([
        jnp.concatenate([enc_w2[:, :, ky, 2 * dx].T, enc_w2[:, :, ky, 2 * dx + 1].T], axis=0)
        for ky in range(4) for dx in range(2)]).astype(BF16)
    h2 = pl.pallas_call(
        _enc2_kernel,
        grid=(B,),
        in_specs=[_per_img((B, 33, 33, 512)), _full((8, 256, 256)), _full((1, 256))],
        out_specs=_per_img((B, 32, 32, 256)),
        out_shape=jax.ShapeDtypeStruct((B, 32, 32, 256), F32),
        compiler_params=_PAR,
    )(X2b, w2stack, enc_b2[None, :])

    w3stack = jnp.stack([enc_w3[:, :, ky, kx].T
                         for ky in range(3) for kx in range(3)]).astype(BF16)
    idx2, qb, lossp = pl.pallas_call(
        _enc3_vq_kernel,
        grid=(B,),
        in_specs=[_per_img((B, 32, 32, 256)), _full((9, 256, 256)), _full((1, 256)),
                  _full((256, 512)), _full((512, 256))],
        out_specs=(pl.BlockSpec((1, 1, 1024), lambda b: (b, 0, 0)),
                   _per_img((B, 32, 32, 256)),
                   pl.BlockSpec((1, 1, 256), lambda b: (b, 0, 0))),
        out_shape=(jax.ShapeDtypeStruct((B, 1, 1024), jnp.int32),
                   jax.ShapeDtypeStruct((B, 32, 32, 256), BF16),
                   jax.ShapeDtypeStruct((B, 1, 256), F32)),
        compiler_params=_PAR,
    )(h2, w3stack, enc_b3[None, :], emb.T, emb)

    dictionary_loss = jnp.sum(lossp) / F32(B * 32 * 32 * 256)
    idx = idx2.reshape(B, 32, 32)

    def tap9(w):
        return jnp.stack([w[:, :, ky, kx].T for ky in range(3) for kx in range(3)]).astype(BF16)

    h = pl.pallas_call(
        _dec_res_kernel,
        grid=(B,),
        in_specs=[_per_img((B, 32, 32, 256)),
                  _full((9, 256, 256)), _full((1, 256)),
                  _full((9, 256, 256)), _full((1, 256)),
                  _full((256, 256)), _full((1, 256)),
                  _full((9, 256, 256)), _full((1, 256)),
                  _full((256, 256)), _full((1, 256))],
        out_specs=_per_img((B, 32, 32, 256)),
        out_shape=jax.ShapeDtypeStruct((B, 32, 32, 256), BF16),
        compiler_params=_PAR,
    )(qb, tap9(dec_w1), dec_b1[None, :],
      tap9(r1_w1), r1_b1[None, :], r1_w2[:, :, 0, 0].T.astype(BF16), r1_b2[None, :],
      tap9(r2_w1), r2_b1[None, :], r2_w2[:, :, 0, 0].T.astype(BF16), r2_b2[None, :])

    wt1 = jnp.stack([dect_w1[:, :, ky, kx].astype(BF16)
                     for py in range(2) for px in range(2)
                     for (_sy, ky) in _PH[py] for (_sx, kx) in _PH[px]])
    hp = pl.pallas_call(
        _dect1_kernel,
        grid=(B,),
        in_specs=[_per_img((B, 32, 32, 256)), _full((16, 256, 256)), _full((1, 256))],
        out_specs=pl.BlockSpec((1, 4, 32, 32, 256), lambda b: (b, 0, 0, 0, 0)),
        out_shape=jax.ShapeDtypeStruct((B, 4, 32, 32, 256), BF16),
        compiler_params=_PAR,
    )(h, wt1, dect_b1[None, :])
    h64 = hp.reshape(B, 2, 2, 32, 32, 256).transpose(0, 3, 1, 4, 2, 5).reshape(B, 64, 64, 256)

    kmap = ({0: 1, -1: 3}, {1: 0, 0: 2})
    cols = []
    zero3 = jnp.zeros((256, 3), F32)
    for sy in (-1, 0, 1):
        for sx in (-1, 0, 1):
            blocks = []
            for py in range(2):
                for px in range(2):
                    ky = kmap[py].get(sy)
                    kx = kmap[px].get(sx)
                    if ky is None or kx is None:
                        blocks.append(zero3)
                    else:
                        blocks.append(dect_w2[:, :, ky, kx])
            w12 = jnp.concatenate(blocks, axis=1)
            cols.append(jnp.pad(w12, ((0, 0), (0, 4))))
    wt2 = jnp.stack(cols).astype(BF16)
    b16 = jnp.pad(jnp.tile(dect_b2, 4), (0, 4))[None, :]
    y = pl.pallas_call(
        _dect2_kernel,
        grid=(B,),
        in_specs=[_per_img((B, 64, 64, 256)), _full((9, 256, 16)), _full((1, 16))],
        out_specs=_per_img((B, 64, 64, 16)),
        out_shape=jax.ShapeDtypeStruct((B, 64, 64, 16), F32),
        compiler_params=_PAR,
    )(h64, wt2, b16)
    x_hat = (y[..., :12].reshape(B, 64, 64, 2, 2, 3)
             .transpose(0, 1, 3, 2, 4, 5).reshape(B, 128, 128, 3)
             .transpose(0, 3, 1, 2))

    return x_hat, dictionary_loss, dictionary_loss, idx


# speedup vs baseline: 1.1897x; 1.1897x over previous
import jax
import jax.numpy as jnp
from jax import lax
from jax.experimental import pallas as pl
from jax.experimental.pallas import tpu as pltpu

F32 = jnp.float32
BF16 = jnp.bfloat16
HIGHEST = lax.Precision.HIGHEST

_PAR = pltpu.CompilerParams(dimension_semantics=("parallel",))


def _shift2d(v, sy, sx):
    H, W, C = v.shape
    core = v[max(0, sy):H + min(0, sy), max(0, sx):W + min(0, sx), :]
    return jnp.pad(core, ((max(0, -sy), max(0, sy)),
                          (max(0, -sx), max(0, sx)), (0, 0)))


def _enc_vq_kernel(l_ref, w1_ref, b1_ref, w2_ref, b2_ref, w3_ref, b3_ref,
                   embT_ref, emb_ref, idx_ref, q_ref, loss_ref):
    planes = []
    for jy in range(2):
        for jx in range(2):
            p = jy * 2 + jx
            Lp = l_ref[0, p].reshape(1089, 48).astype(BF16)
            h = jnp.dot(Lp, w1_ref[...], preferred_element_type=F32)
            h = jnp.maximum(h + b1_ref[...], 0.0).reshape(33, 33, 128)
            oy = 2 * lax.broadcasted_iota(jnp.int32, (33, 33, 1), 0) + jy - 1
            ox = 2 * lax.broadcasted_iota(jnp.int32, (33, 33, 1), 1) + jx - 1
            valid = (oy >= 0) & (oy <= 63) & (ox >= 0) & (ox <= 63)
            planes.append(jnp.where(valid, h, 0.0))
    X = jnp.concatenate(planes, axis=-1)

    acc = None
    i = 0
    for ky in range(4):
        dy, jy = divmod(ky, 2)
        for dx in range(2):
            t = X[dy:dy + 32, dx:dx + 32, jy * 256:(jy + 1) * 256]
            p = jnp.dot(t.reshape(1024, 256).astype(BF16), w2_ref[i],
                        preferred_element_type=F32)
            acc = p if acc is None else acc + p
            i += 1
    h2 = jnp.maximum(acc + b2_ref[...], 0.0).reshape(32, 32, 256)

    Xp = jnp.pad(h2, ((1, 1), (1, 1), (0, 0)))
    acc = None
    for i in range(9):
        ky, kx = divmod(i, 3)
        t = Xp[ky:ky + 32, kx:kx + 32, :].reshape(1024, 256).astype(BF16)
        p = jnp.dot(t, w3_ref[i], preferred_element_type=F32)
        acc = p if acc is None else acc + p
    z = acc + b3_ref[...]

    A = jnp.sum(z * z, axis=1, keepdims=True)
    embT = embT_ref[...]
    Brow = jnp.sum(embT * embT, axis=0, keepdims=True)
    C = jnp.dot(z.astype(BF16), embT.astype(BF16),
                preferred_element_type=F32)
    dists = (A + Brow) - 2.0 * C
    m = jnp.min(dists, axis=1, keepdims=True)
    iota = lax.broadcasted_iota(jnp.int32, (1024, 512), 1)
    am = jnp.min(jnp.where(dists == m, iota, 512), axis=1).astype(jnp.int32)
    idx_ref[0] = am[None, :]

    onehot = (iota == am[:, None]).astype(F32)
    q = jnp.dot(onehot, emb_ref[...], preferred_element_type=F32,
                precision=HIGHEST)
    q_ref[0] = q.reshape(32, 32, 256).astype(BF16)
    diff = q - z
    loss_ref[0] = jnp.sum(diff * diff, axis=0, keepdims=True)


_PH = (((0, 1), (-1, 3)), ((1, 0), (0, 2)))


def _conv3x3_bf16(v, w_ref, b):
    vp = jnp.pad(v, ((1, 1), (1, 1), (0, 0)))
    acc = None
    for i in range(9):
        ky, kx = divmod(i, 3)
        t = vp[ky:ky + 32, kx:kx + 32, :].reshape(1024, 256)
        p = jnp.dot(t, w_ref[i], preferred_element_type=F32)
        acc = p if acc is None else acc + p
    return acc + b


def _decoder_kernel(x_ref, wd_ref, bd_ref, w1a_ref, b1a_ref, w1b_ref, b1b_ref,
                    w2a_ref, b2a_ref, w2b_ref, b2b_ref,
                    wt1_ref, bt1_ref, wt2_ref, bt2_ref, o_ref):
    X = x_ref[0]
    h = _conv3x3_bf16(X, wd_ref, bd_ref[...])
    for wa, ba, wb, bb in ((w1a_ref, b1a_ref, w1b_ref, b1b_ref),
                           (w2a_ref, b2a_ref, w2b_ref, b2b_ref)):
        r = jnp.maximum(h, 0.0).astype(BF16).reshape(32, 32, 256)
        t = _conv3x3_bf16(r, wa, ba[...])
        t = jnp.maximum(t, 0.0).astype(BF16)
        t = jnp.dot(t, wb[...], preferred_element_type=F32) + bb[...]
        h = h + t
    hb = h.astype(BF16).reshape(32, 32, 256)

    HP = [[None, None], [None, None]]
    i = 0
    for py in range(2):
        for px in range(2):
            acc = None
            for (sy, _ky) in _PH[py]:
                for (sx, _kx) in _PH[px]:
                    t = _shift2d(hb, sy, sx).reshape(1024, 256)
                    p = jnp.dot(t, wt1_ref[i], preferred_element_type=F32)
                    acc = p if acc is None else acc + p
                    i += 1
            HP[py][px] = (jnp.maximum(acc + bt1_ref[...], 0.0)
                          .astype(BF16).reshape(32, 32, 256))

    for py in range(2):
        for px in range(2):
            acc = None
            for i in range(9):
                sy, sx = divmod(i, 3)
                sy -= 1
                sx -= 1
                phy, offy = (py + sy) % 2, (py + sy) // 2
                phx, offx = (px + sx) % 2, (px + sx) // 2
                t = _shift2d(HP[phy][phx], offy, offx).reshape(1024, 256)
                p = jnp.dot(t, wt2_ref[i], preferred_element_type=F32)
                acc = p if acc is None else acc + p
            y = jax.nn.sigmoid(acc + bt2_ref[...])
            o_ref[0, py, px] = y.reshape(32, 32, 16)


def _full(shape):
    nd = len(shape)
    return pl.BlockSpec(shape, lambda b: (0,) * nd)


def _per_img(shape):
    nd = len(shape)
    return pl.BlockSpec((1,) + shape[1:], lambda b: (b,) + (0,) * (nd - 1))


def kernel(x, emb, enc_w1, enc_b1, enc_w2, enc_b2, enc_w3, enc_b3,
           dec_w1, dec_b1, r1_w1, r1_b1, r1_w2, r1_b2,
           r2_w1, r2_b1, r2_w2, r2_b2, dect_w1, dect_b1, dect_w2, dect_b2):
    B = x.shape[0]

    xp = jnp.pad(x.transpose(0, 2, 3, 1), ((0, 0), (4, 4), (4, 4), (0, 0)))
    x4 = xp.reshape(B, 34, 4, 34, 4, 3).transpose(0, 1, 3, 2, 4, 5)
    wparts = [enc_w1[:, :, ky, kx].T for ky in range(4) for kx in range(4)]
    phs = []
    for jy in range(2):
        for jx in range(2):
            parts = []
            for ky in range(4):
                uy, ry = divmod(2 * jy + ky + 1, 4)
                for kx in range(4):
                    ux, rx = divmod(2 * jx + kx + 1, 4)
                    parts.append(x4[:, uy:uy + 33, ux:ux + 33, ry, rx, :])
            phs.append(jnp.concatenate(parts, axis=-1))
    L = jnp.stack(phs, axis=1)
    W48 = jnp.concatenate(wparts, axis=0).astype(BF16)

    w2stack = jnp.stack([
        jnp.concatenate([enc_w2[:, :, ky, 2 * dx].T, enc_w2[:, :, ky, 2 * dx + 1].T], axis=0)
        for ky in range(4) for dx in range(2)]).astype(BF16)
    w3stack = jnp.stack([enc_w3[:, :, ky, kx].T
                         for ky in range(3) for kx in range(3)]).astype(BF16)

    idx2, qb, lossp = pl.pallas_call(
        _enc_vq_kernel,
        grid=(B,),
        in_specs=[_per_img((B, 4, 33, 33, 48)), _full((48, 128)), _full((1, 128)),
                  _full((8, 256, 256)), _full((1, 256)),
                  _full((9, 256, 256)), _full((1, 256)),
                  _full((256, 512)), _full((512, 256))],
        out_specs=(pl.BlockSpec((1, 1, 1024), lambda b: (b, 0, 0)),
                   _per_img((B, 32, 32, 256)),
                   pl.BlockSpec((1, 1, 256), lambda b: (b, 0, 0))),
        out_shape=(jax.ShapeDtypeStruct((B, 1, 1024), jnp.int32),
                   jax.ShapeDtypeStruct((B, 32, 32, 256), BF16),
                   jax.ShapeDtypeStruct((B, 1, 256), F32)),
        compiler_params=_PAR,
    )(L, W48, enc_b1[None, :], w2stack, enc_b2[None, :], w3stack, enc_b3[None, :],
      emb.T, emb)

    dictionary_loss = jnp.sum(lossp) / F32(B * 32 * 32 * 256)
    idx = idx2.reshape(B, 32, 32)

    def tap9(w):
        return jnp.stack([w[:, :, ky, kx].T for ky in range(3) for kx in range(3)]).astype(BF16)

    wt1 = jnp.stack([dect_w1[:, :, ky, kx].astype(BF16)
                     for py in range(2) for px in range(2)
                     for (_sy, ky) in _PH[py] for (_sx, kx) in _PH[px]])

    kmap = ({0: 1, -1: 3}, {1: 0, 0: 2})
    cols = []
    zero3 = jnp.zeros((256, 3), F32)
    for sy in (-1, 0, 1):
        for sx in (-1, 0, 1):
            blocks = []
            for qy in range(2):
                for qx in range(2):
                    ky = kmap[qy].get(sy)
                    kx = kmap[qx].get(sx)
                    blocks.append(zero3 if ky is None or kx is None
                                  else dect_w2[:, :, ky, kx])
            w12 = jnp.concatenate(blocks, axis=1)
            cols.append(jnp.pad(w12, ((0, 0), (0, 4))))
    wt2 = jnp.stack(cols).astype(BF16)
    b16 = jnp.pad(jnp.tile(dect_b2, 4), (0, 4))[None, :]

    y = pl.pallas_call(
        _decoder_kernel,
        grid=(B,),
        in_specs=[_per_img((B, 32, 32, 256)),
                  _full((9, 256, 256)), _full((1, 256)),
                  _full((9, 256, 256)), _full((1, 256)),
                  _full((256, 256)), _full((1, 256)),
                  _full((9, 256, 256)), _full((1, 256)),
                  _full((256, 256)), _full((1, 256)),
                  _full((16, 256, 256)), _full((1, 256)),
                  _full((9, 256, 16)), _full((1, 16))],
        out_specs=pl.BlockSpec((1, 2, 2, 32, 32, 16), lambda b: (b, 0, 0, 0, 0, 0)),
        out_shape=jax.ShapeDtypeStruct((B, 2, 2, 32, 32, 16), F32),
        compiler_params=_PAR,
    )(qb, tap9(dec_w1), dec_b1[None, :],
      tap9(r1_w1), r1_b1[None, :], r1_w2[:, :, 0, 0].T.astype(BF16), r1_b2[None, :],
      tap9(r2_w1), r2_b1[None, :], r2_w2[:, :, 0, 0].T.astype(BF16), r2_b2[None, :],
      wt1, dect_b1[None, :], wt2, b16)

    x_hat = (y[..., :12].reshape(B, 2, 2, 32, 32, 2, 2, 3)
             .transpose(0, 3, 1, 5, 4, 2, 6, 7).reshape(B, 128, 128, 3)
             .transpose(0, 3, 1, 2))

    return x_hat, dictionary_loss, dictionary_loss, idx


# speedup vs baseline: 2.3131x; 1.9443x over previous
import jax
import jax.numpy as jnp
from jax import lax
from jax.experimental import pallas as pl
from jax.experimental.pallas import tpu as pltpu

F32 = jnp.float32
BF16 = jnp.bfloat16
HIGHEST = lax.Precision.HIGHEST

_PAR = pltpu.CompilerParams(dimension_semantics=("parallel",))


def _shift2d(v, sy, sx):
    H, W, C = v.shape
    core = v[max(0, sy):H + min(0, sy), max(0, sx):W + min(0, sx), :]
    return jnp.pad(core, ((max(0, -sy), max(0, sy)),
                          (max(0, -sx), max(0, sx)), (0, 0)))


def _enc_vq_kernel(l_ref, w1_ref, b1_ref, w2_ref, b2_ref, w3_ref, b3_ref,
                   embT_ref, emb_ref, idx_ref, q_ref, loss_ref):
    planes = []
    L = l_ref[0].reshape(1089, 192).astype(BF16)
    for jy in range(2):
        for jx in range(2):
            p = jy * 2 + jx
            h = jnp.dot(L, w1_ref[p], preferred_element_type=F32)
            h = jnp.maximum(h + b1_ref[...], 0.0).reshape(33, 33, 128)
            oy = 2 * lax.broadcasted_iota(jnp.int32, (33, 33, 1), 0) + jy - 1
            ox = 2 * lax.broadcasted_iota(jnp.int32, (33, 33, 1), 1) + jx - 1
            valid = (oy >= 0) & (oy <= 63) & (ox >= 0) & (ox <= 63)
            planes.append(jnp.where(valid, h, 0.0))
    X = jnp.concatenate(planes, axis=-1)

    acc = None
    i = 0
    for ky in range(4):
        dy, jy = divmod(ky, 2)
        for dx in range(2):
            t = X[dy:dy + 32, dx:dx + 32, jy * 256:(jy + 1) * 256]
            p = jnp.dot(t.reshape(1024, 256).astype(BF16), w2_ref[i],
                        preferred_element_type=F32)
            acc = p if acc is None else acc + p
            i += 1
    h2 = jnp.maximum(acc + b2_ref[...], 0.0).reshape(32, 32, 256)

    Xp = jnp.pad(h2, ((1, 1), (1, 1), (0, 0)))
    acc = None
    for i in range(9):
        ky, kx = divmod(i, 3)
        t = Xp[ky:ky + 32, kx:kx + 32, :].reshape(1024, 256).astype(BF16)
        p = jnp.dot(t, w3_ref[i], preferred_element_type=F32)
        acc = p if acc is None else acc + p
    z = acc + b3_ref[...]

    A = jnp.sum(z * z, axis=1, keepdims=True)
    embT = embT_ref[...]
    Brow = jnp.sum(embT * embT, axis=0, keepdims=True)
    C = jnp.dot(z.astype(BF16), embT.astype(BF16),
                preferred_element_type=F32)
    dists = (A + Brow) - 2.0 * C
    m = jnp.min(dists, axis=1, keepdims=True)
    iota = lax.broadcasted_iota(jnp.int32, (1024, 512), 1)
    am = jnp.min(jnp.where(dists == m, iota, 512), axis=1).astype(jnp.int32)
    idx_ref[0] = am[None, :]

    onehot = (iota == am[:, None]).astype(F32)
    q = jnp.dot(onehot, emb_ref[...], preferred_element_type=F32,
                precision=HIGHEST)
    q_ref[0] = q.reshape(32, 32, 256).astype(BF16)
    diff = q - z
    loss_ref[0] = jnp.sum(diff * diff, axis=0, keepdims=True)


_PH = (((0, 1), (-1, 3)), ((1, 0), (0, 2)))


def _conv3x3_bf16(v, w_ref, b):
    vp = jnp.pad(v, ((1, 1), (1, 1), (0, 0)))
    acc = None
    for i in range(9):
        ky, kx = divmod(i, 3)
        t = vp[ky:ky + 32, kx:kx + 32, :].reshape(1024, 256)
        p = jnp.dot(t, w_ref[i], preferred_element_type=F32)
        acc = p if acc is None else acc + p
    return acc + b


def _decoder_kernel(x_ref, wd_ref, bd_ref, w1a_ref, b1a_ref, w1b_ref, b1b_ref,
                    w2a_ref, b2a_ref, w2b_ref, b2b_ref,
                    wt1_ref, bt1_ref, wt2_ref, bt2_ref, o_ref):
    X = x_ref[0]
    h = _conv3x3_bf16(X, wd_ref, bd_ref[...])
    for wa, ba, wb, bb in ((w1a_ref, b1a_ref, w1b_ref, b1b_ref),
                           (w2a_ref, b2a_ref, w2b_ref, b2b_ref)):
        r = jnp.maximum(h, 0.0).astype(BF16).reshape(32, 32, 256)
        t = _conv3x3_bf16(r, wa, ba[...])
        t = jnp.maximum(t, 0.0).astype(BF16)
        t = jnp.dot(t, wb[...], preferred_element_type=F32) + bb[...]
        h = h + t
    hb = h.astype(BF16).reshape(32, 32, 256)

    HP = [[None, None], [None, None]]
    i = 0
    for py in range(2):
        for px in range(2):
            acc = None
            for (sy, _ky) in _PH[py]:
                for (sx, _kx) in _PH[px]:
                    t = _shift2d(hb, sy, sx).reshape(1024, 256)
                    p = jnp.dot(t, wt1_ref[i], preferred_element_type=F32)
                    acc = p if acc is None else acc + p
                    i += 1
            HP[py][px] = (jnp.maximum(acc + bt1_ref[...], 0.0)
                          .astype(BF16).reshape(32, 32, 256))

    for py in range(2):
        for px in range(2):
            acc = None
            for i in range(9):
                sy, sx = divmod(i, 3)
                sy -= 1
                sx -= 1
                phy, offy = (py + sy) % 2, (py + sy) // 2
                phx, offx = (px + sx) % 2, (px + sx) // 2
                t = _shift2d(HP[phy][phx], offy, offx).reshape(1024, 256)
                p = jnp.dot(t, wt2_ref[i], preferred_element_type=F32)
                acc = p if acc is None else acc + p
            y = jax.nn.sigmoid(acc + bt2_ref[...])
            o_ref[0, py, px] = y.reshape(32, 32, 16)


def _full(shape):
    nd = len(shape)
    return pl.BlockSpec(shape, lambda b: (0,) * nd)


def _per_img(shape):
    nd = len(shape)
    return pl.BlockSpec((1,) + shape[1:], lambda b: (b,) + (0,) * (nd - 1))


def kernel(x, emb, enc_w1, enc_b1, enc_w2, enc_b2, enc_w3, enc_b3,
           dec_w1, dec_b1, r1_w1, r1_b1, r1_w2, r1_b2,
           r2_w1, r2_b1, r2_w2, r2_b2, dect_w1, dect_b1, dect_w2, dect_b2):
    B = x.shape[0]

    xp = jnp.pad(x.transpose(0, 2, 3, 1), ((0, 0), (4, 4), (4, 4), (0, 0)))
    x4 = (xp.reshape(B, 34, 4, 34, 4, 3).transpose(0, 1, 3, 2, 4, 5)
          .reshape(B, 34, 34, 48))
    L = jnp.concatenate([x4[:, uy:uy + 33, ux:ux + 33, :]
                         for uy in range(2) for ux in range(2)], axis=-1)
    zrow = jnp.zeros((3, 128), F32)
    wphs = []
    for jy in range(2):
        for jx in range(2):
            rows = []
            for uy in range(2):
                for ux in range(2):
                    for ry in range(4):
                        ky = 4 * uy + ry - 2 * jy - 1
                        for rx in range(4):
                            kx = 4 * ux + rx - 2 * jx - 1
                            rows.append(enc_w1[:, :, ky, kx].T
                                        if 0 <= ky < 4 and 0 <= kx < 4 else zrow)
            wphs.append(jnp.concatenate(rows, axis=0))
    W48 = jnp.stack(wphs).astype(BF16)

    w2stack = jnp.stack([
        jnp.concatenate([enc_w2[:, :, ky, 2 * dx].T, enc_w2[:, :, ky, 2 * dx + 1].T], axis=0)
        for ky in range(4) for dx in range(2)]).astype(BF16)
    w3stack = jnp.stack([enc_w3[:, :, ky, kx].T
                         for ky in range(3) for kx in range(3)]).astype(BF16)

    idx2, qb, lossp = pl.pallas_call(
        _enc_vq_kernel,
        grid=(B,),
        in_specs=[_per_img((B, 33, 33, 192)), _full((4, 192, 128)), _full((1, 128)),
                  _full((8, 256, 256)), _full((1, 256)),
                  _full((9, 256, 256)), _full((1, 256)),
                  _full((256, 512)), _full((512, 256))],
        out_specs=(pl.BlockSpec((1, 1, 1024), lambda b: (b, 0, 0)),
                   _per_img((B, 32, 32, 256)),
                   pl.BlockSpec((1, 1, 256), lambda b: (b, 0, 0))),
        out_shape=(jax.ShapeDtypeStruct((B, 1, 1024), jnp.int32),
                   jax.ShapeDtypeStruct((B, 32, 32, 256), BF16),
                   jax.ShapeDtypeStruct((B, 1, 256), F32)),
        compiler_params=_PAR,
    )(L, W48, enc_b1[None, :], w2stack, enc_b2[None, :], w3stack, enc_b3[None, :],
      emb.T, emb)

    dictionary_loss = jnp.sum(lossp) / F32(B * 32 * 32 * 256)
    idx = idx2.reshape(B, 32, 32)

    def tap9(w):
        return jnp.stack([w[:, :, ky, kx].T for ky in range(3) for kx in range(3)]).astype(BF16)

    wt1 = jnp.stack([dect_w1[:, :, ky, kx].astype(BF16)
                     for py in range(2) for px in range(2)
                     for (_sy, ky) in _PH[py] for (_sx, kx) in _PH[px]])

    kmap = ({0: 1, -1: 3}, {1: 0, 0: 2})
    cols = []
    zero3 = jnp.zeros((256, 3), F32)
    for sy in (-1, 0, 1):
        for sx in (-1, 0, 1):
            blocks = []
            for qy in range(2):
                for qx in range(2):
                    ky = kmap[qy].get(sy)
                    kx = kmap[qx].get(sx)
                    blocks.append(zero3 if ky is None or kx is None
                                  else dect_w2[:, :, ky, kx])
            w12 = jnp.concatenate(blocks, axis=1)
            cols.append(jnp.pad(w12, ((0, 0), (0, 4))))
    wt2 = jnp.stack(cols).astype(BF16)
    b16 = jnp.pad(jnp.tile(dect_b2, 4), (0, 4))[None, :]

    y = pl.pallas_call(
        _decoder_kernel,
        grid=(B,),
        in_specs=[_per_img((B, 32, 32, 256)),
                  _full((9, 256, 256)), _full((1, 256)),
                  _full((9, 256, 256)), _full((1, 256)),
                  _full((256, 256)), _full((1, 256)),
                  _full((9, 256, 256)), _full((1, 256)),
                  _full((256, 256)), _full((1, 256)),
                  _full((16, 256, 256)), _full((1, 256)),
                  _full((9, 256, 16)), _full((1, 16))],
        out_specs=pl.BlockSpec((1, 2, 2, 32, 32, 16), lambda b: (b, 0, 0, 0, 0, 0)),
        out_shape=jax.ShapeDtypeStruct((B, 2, 2, 32, 32, 16), F32),
        compiler_params=_PAR,
    )(qb, tap9(dec_w1), dec_b1[None, :],
      tap9(r1_w1), r1_b1[None, :], r1_w2[:, :, 0, 0].T.astype(BF16), r1_b2[None, :],
      tap9(r2_w1), r2_b1[None, :], r2_w2[:, :, 0, 0].T.astype(BF16), r2_b2[None, :],
      wt1, dect_b1[None, :], wt2, b16)

    x_hat = (y[..., :12].reshape(B, 2, 2, 32, 32, 2, 2, 3)
             .transpose(0, 3, 1, 5, 4, 2, 6, 7).reshape(B, 128, 128, 3)
             .transpose(0, 3, 1, 2))

    return x_hat, dictionary_loss, dictionary_loss, idx
